# Initial kernel scaffold; baseline (speedup 1.0000x reference)
import jax
import jax.numpy as jnp
from jax.experimental import pallas as pl


def kernel(x, edge_index, edge_weight, W_rel1, b_rel1, W_root1, W_rel2, b_rel2, W_root2):
    raise NotImplementedError("write your pallas kernel here")


# speedup vs baseline: 13.2348x; 13.2348x over previous
import functools

import jax
import jax.numpy as jnp
from jax import lax
from jax.experimental import pallas as pl
from jax.experimental.pallas import tpu as pltpu
from jax.experimental.pallas import tpu_sc as plsc

N_NODES = 10000
N_EDGES = 320000
D_IN = 128
D_HID = 16

CHUNK = 80
ROWS = 4096
ROWS_L1 = ROWS // 32
ROWS_L2 = ROWS // 16
EPAD = ROWS * CHUNK


def _mm_kernel(x_ref, wr_ref, wo_ref, y1_ref, r1_ref):
    xb = x_ref[...]
    dn = (((1,), (1,)), ((), ()))
    y1_ref[...] = lax.dot_general(xb, wr_ref[...], dn,
                                  preferred_element_type=jnp.float32)
    r1_ref[...] = lax.dot_general(xb, wo_ref[...], dn,
                                  preferred_element_type=jnp.float32)


def _proj1(x, W_rel1, W_root1):
    blk = 1000
    return pl.pallas_call(
        _mm_kernel,
        grid=(N_NODES // blk,),
        in_specs=[
            pl.BlockSpec((blk, D_IN), lambda i: (i, 0)),
            pl.BlockSpec((D_HID, D_IN), lambda i: (0, 0)),
            pl.BlockSpec((D_HID, D_IN), lambda i: (0, 0)),
        ],
        out_specs=[
            pl.BlockSpec((blk, D_HID), lambda i: (i, 0)),
            pl.BlockSpec((blk, D_HID), lambda i: (i, 0)),
        ],
        out_shape=[
            jax.ShapeDtypeStruct((N_NODES, D_HID), jnp.float32),
            jax.ShapeDtypeStruct((N_NODES, D_HID), jnp.float32),
        ],
    )(x, W_rel1, W_root1)


def _sc_aggr1_body(y1, srcR, dstR, ewR, part,
                   src_v, dst_v, ew_v, rows, zbuf, aggr, sem):
    c = lax.axis_index("c")
    s = lax.axis_index("s")
    wid = c * 16 + s
    base = wid * ROWS_L1

    pltpu.sync_copy(srcR.at[pl.ds(base, ROWS_L1)], src_v)
    pltpu.sync_copy(dstR.at[pl.ds(base, ROWS_L1)], dst_v)
    pltpu.sync_copy(ewR.at[pl.ds(base, ROWS_L1)], ew_v)

    def _z(i, _):
        zbuf[i, :] = jnp.zeros((16,), jnp.float32)
        return 0
    lax.fori_loop(0, 640, _z, 0)

    @pl.when(s < 15)
    def _():
        pltpu.sync_copy(zbuf, aggr.at[pl.ds(s * 640, 640)])

    @pl.when(s == 15)
    def _():
        pltpu.sync_copy(zbuf.at[pl.ds(0, 400)], aggr.at[pl.ds(9600, 400)])

    plsc.subcore_barrier()

    def _chunk(j, _):
        pltpu.async_copy(y1.at[src_v.at[j]], rows, sem).wait()
        for k in range(CHUNK // 16):
            w = ew_v[j, pl.ds(16 * k, 16)]
            for e in range(16):
                rows[16 * k + e, :] = rows[16 * k + e, :] * w[e]
        pltpu.sync_copy(rows, aggr.at[dst_v.at[j]], add=True)
        return 0
    lax.fori_loop(0, ROWS_L1, _chunk, 0)
    plsc.subcore_barrier()

    @pl.when(s < 15)
    def _():
        pltpu.sync_copy(aggr.at[pl.ds(s * 640, 640)], zbuf)
        pltpu.sync_copy(zbuf, part.at[c, pl.ds(s * 640, 640)])

    @pl.when(s == 15)
    def _():
        pltpu.sync_copy(aggr.at[pl.ds(9600, 400)], zbuf.at[pl.ds(0, 400)])
        pltpu.sync_copy(zbuf.at[pl.ds(0, 400)], part.at[c, pl.ds(9600, 400)])


def _sc_aggr1(y1, srcR, dstR, ewR):
    mesh = plsc.VectorSubcoreMesh(core_axis_name="c", subcore_axis_name="s")
    return pl.kernel(
        _sc_aggr1_body,
        out_type=jax.ShapeDtypeStruct((2, N_NODES, D_HID), jnp.float32),
        mesh=mesh,
        compiler_params=pltpu.CompilerParams(use_tc_tiling_on_sc=False, needs_layout_passes=False),
        scratch_types=[
            pltpu.VMEM((ROWS_L1, CHUNK), jnp.int32),
            pltpu.VMEM((ROWS_L1, CHUNK), jnp.int32),
            pltpu.VMEM((ROWS_L1, CHUNK), jnp.float32),
            pltpu.VMEM((CHUNK, D_HID), jnp.float32),
            pltpu.VMEM((640, D_HID), jnp.float32),
            pltpu.VMEM_SHARED((N_NODES, D_HID), jnp.float32),
            pltpu.SemaphoreType.DMA,
        ],
    )(y1, srcR, dstR, ewR)


def _mid_kernel(p0_ref, p1_ref, r1_ref, b1_ref, wr2_ref, wo2_ref, b2_ref,
                y2_ref, r2pb_ref):
    h = jnp.maximum(
        p0_ref[...] + p1_ref[...] + r1_ref[...] + b1_ref[...], 0.0)
    y2_ref[...] = jnp.sum(h * wr2_ref[...], axis=1, keepdims=True)
    r2pb_ref[...] = (jnp.sum(h * wo2_ref[...], axis=1, keepdims=True)
                     + b2_ref[...])


def _mid(p0, p1, r1, b1, wr2, wo2, b2):
    blk = 1000
    return pl.pallas_call(
        _mid_kernel,
        grid=(N_NODES // blk,),
        in_specs=[
            pl.BlockSpec((blk, D_HID), lambda i: (i, 0)),
            pl.BlockSpec((blk, D_HID), lambda i: (i, 0)),
            pl.BlockSpec((blk, D_HID), lambda i: (i, 0)),
            pl.BlockSpec((1, D_HID), lambda i: (0, 0)),
            pl.BlockSpec((1, D_HID), lambda i: (0, 0)),
            pl.BlockSpec((1, D_HID), lambda i: (0, 0)),
            pl.BlockSpec((1, 1), lambda i: (0, 0)),
        ],
        out_specs=[
            pl.BlockSpec((blk, 1), lambda i: (i, 0)),
            pl.BlockSpec((blk, 1), lambda i: (i, 0)),
        ],
        out_shape=[
            jax.ShapeDtypeStruct((N_NODES, 1), jnp.float32),
            jax.ShapeDtypeStruct((N_NODES, 1), jnp.float32),
        ],
    )(p0, p1, r1, b1, wr2, wo2, b2)


def _sc_aggr2_body(y2, r2pb, srcR, dstR, ewR, out,
                   y2_v, src_v, dst_v, ew_v, prod, ibuf, aggr, sem):
    c = lax.axis_index("c")
    s = lax.axis_index("s")

    @pl.when(c == 0)
    def _():
        pltpu.sync_copy(y2, y2_v)
        pltpu.sync_copy(srcR.at[pl.ds(s * ROWS_L2, ROWS_L2)], src_v)
        pltpu.sync_copy(dstR.at[pl.ds(s * ROWS_L2, ROWS_L2)], dst_v)
        pltpu.sync_copy(ewR.at[pl.ds(s * ROWS_L2, ROWS_L2)], ew_v)

        @pl.when(s < 15)
        def _():
            pltpu.sync_copy(r2pb.at[pl.ds(s * 640, 640)], ibuf)
            pltpu.sync_copy(ibuf, aggr.at[pl.ds(s * 640, 640)])

        @pl.when(s == 15)
        def _():
            pltpu.sync_copy(r2pb.at[pl.ds(9600, 400)], ibuf.at[pl.ds(0, 400)])
            pltpu.sync_copy(ibuf.at[pl.ds(0, 400)], aggr.at[pl.ds(9600, 400)])

        plsc.subcore_barrier()

        def _chunk(j, _):
            for k in range(CHUNK // 16):
                idx = src_v[j, pl.ds(16 * k, 16)]
                vals = plsc.load_gather(y2_v, [idx])
                w = ew_v[j, pl.ds(16 * k, 16)]
                prod[pl.ds(16 * k, 16)] = vals * w
            pltpu.sync_copy(prod, aggr.at[dst_v.at[j]], add=True)
            return 0
        lax.fori_loop(0, ROWS_L2, _chunk, 0)
        plsc.subcore_barrier()

        @pl.when(s < 15)
        def _():
            pltpu.sync_copy(aggr.at[pl.ds(s * 640, 640)], ibuf)
            pltpu.sync_copy(ibuf, out.at[pl.ds(s * 640, 640)])

        @pl.when(s == 15)
        def _():
            pltpu.sync_copy(aggr.at[pl.ds(9600, 400)], ibuf.at[pl.ds(0, 400)])
            pltpu.sync_copy(ibuf.at[pl.ds(0, 400)], out.at[pl.ds(9600, 400)])


def _sc_aggr2(y2, r2pb, srcR, dstR, ewR):
    mesh = plsc.VectorSubcoreMesh(core_axis_name="c", subcore_axis_name="s")
    return pl.kernel(
        _sc_aggr2_body,
        out_type=jax.ShapeDtypeStruct((N_NODES,), jnp.float32),
        mesh=mesh,
        compiler_params=pltpu.CompilerParams(use_tc_tiling_on_sc=False, needs_layout_passes=False),
        scratch_types=[
            pltpu.VMEM((N_NODES,), jnp.float32),
            pltpu.VMEM((ROWS_L2, CHUNK), jnp.int32),
            pltpu.VMEM((ROWS_L2, CHUNK), jnp.int32),
            pltpu.VMEM((ROWS_L2, CHUNK), jnp.float32),
            pltpu.VMEM((CHUNK,), jnp.float32),
            pltpu.VMEM((640,), jnp.float32),
            pltpu.VMEM_SHARED((N_NODES,), jnp.float32),
            pltpu.SemaphoreType.DMA,
        ],
    )(y2, r2pb, srcR, dstR, ewR)


def kernel(x, edge_index, edge_weight, W_rel1, b_rel1, W_root1,
           W_rel2, b_rel2, W_root2):
    npad = EPAD - N_EDGES
    ei = edge_index.astype(jnp.int32)
    srcR = jnp.concatenate(
        [ei[0], jnp.zeros((npad,), jnp.int32)]).reshape(ROWS, CHUNK)
    dstR = jnp.concatenate(
        [ei[1], jnp.zeros((npad,), jnp.int32)]).reshape(ROWS, CHUNK)
    ewR = jnp.concatenate(
        [edge_weight, jnp.zeros((npad,), jnp.float32)]).reshape(ROWS, CHUNK)

    y1, r1 = _proj1(x, W_rel1, W_root1)
    part = _sc_aggr1(y1, srcR, dstR, ewR)
    y2, r2pb = _mid(part[0], part[1], r1, b_rel1.reshape(1, D_HID),
                    W_rel2, W_root2, b_rel2.reshape(1, 1))
    out = _sc_aggr2(y2.reshape(N_NODES), r2pb.reshape(N_NODES),
                    srcR, dstR, ewR)
    return out.reshape(N_NODES, 1)


# speedup vs baseline: 20.2362x; 1.5290x over previous
import functools

import jax
import jax.numpy as jnp
from jax import lax
from jax.experimental import pallas as pl
from jax.experimental.pallas import tpu as pltpu
from jax.experimental.pallas import tpu_sc as plsc

N_NODES = 10000
N_EDGES = 320000
D_IN = 128
D_HID = 16

CHUNK = 128
ROWS = 2560
ROWS_L1 = ROWS // 32
ROWS_L2 = ROWS // 16
EPAD = ROWS * CHUNK
NBUF = 4


def _mm_kernel(x_ref, wr_ref, wo_ref, y1_ref, r1_ref):
    xb = x_ref[...]
    dn = (((1,), (1,)), ((), ()))
    y1_ref[...] = lax.dot_general(xb, wr_ref[...], dn,
                                  preferred_element_type=jnp.float32)
    r1_ref[...] = lax.dot_general(xb, wo_ref[...], dn,
                                  preferred_element_type=jnp.float32)


def _proj1(x, W_rel1, W_root1):
    blk = 1000
    return pl.pallas_call(
        _mm_kernel,
        grid=(N_NODES // blk,),
        in_specs=[
            pl.BlockSpec((blk, D_IN), lambda i: (i, 0)),
            pl.BlockSpec((D_HID, D_IN), lambda i: (0, 0)),
            pl.BlockSpec((D_HID, D_IN), lambda i: (0, 0)),
        ],
        out_specs=[
            pl.BlockSpec((blk, D_HID), lambda i: (i, 0)),
            pl.BlockSpec((blk, D_HID), lambda i: (i, 0)),
        ],
        out_shape=[
            jax.ShapeDtypeStruct((N_NODES, D_HID), jnp.float32),
            jax.ShapeDtypeStruct((N_NODES, D_HID), jnp.float32),
        ],
    )(x, W_rel1, W_root1)


def _sc_aggr1_body(y1, srcR, dstR, ewR, part,
                   src_v, dst_v, ew_v, gbuf, sbuf, zbuf, aggr,
                   gsem, ssem):
    c = lax.axis_index("c")
    s = lax.axis_index("s")
    wid = c * 16 + s
    base = wid * ROWS_L1

    pltpu.sync_copy(srcR.at[pl.ds(base, ROWS_L1)], src_v)
    pltpu.sync_copy(dstR.at[pl.ds(base, ROWS_L1)], dst_v)
    pltpu.sync_copy(ewR.at[pl.ds(base, ROWS_L1)], ew_v)

    def _z(i, _):
        zbuf[i, :] = jnp.zeros((16,), jnp.float32)
        return 0
    lax.fori_loop(0, 640, _z, 0)

    @pl.when(s < 15)
    def _():
        pltpu.sync_copy(zbuf, aggr.at[pl.ds(s * 640, 640)])

    @pl.when(s == 15)
    def _():
        pltpu.sync_copy(zbuf.at[pl.ds(0, 400)], aggr.at[pl.ds(9600, 400)])

    plsc.subcore_barrier()

    n_rounds = ROWS_L1 // NBUF
    for jj in range(NBUF):
        pltpu.async_copy(y1.at[src_v.at[jj]], gbuf.at[jj], gsem[jj])

    def _round(r, _):
        for jj in range(NBUF):
            j = r * NBUF + jj
            pltpu.make_async_copy(
                y1.at[src_v.at[j]], gbuf.at[jj], gsem[jj]).wait()

            @pl.when(r > 0)
            def _():
                pltpu.make_async_copy(
                    sbuf.at[jj], aggr.at[dst_v.at[j]], ssem[jj]).wait()

            for k in range(CHUNK // 16):
                w = ew_v[j, pl.ds(16 * k, 16)]
                for e in range(16):
                    sbuf[jj, 16 * k + e, :] = gbuf[jj, 16 * k + e, :] * w[e]

            @pl.when(r < n_rounds - 1)
            def _():
                pltpu.async_copy(
                    y1.at[src_v.at[j + NBUF]], gbuf.at[jj], gsem[jj])

            pltpu.async_copy(
                sbuf.at[jj], aggr.at[dst_v.at[j]], ssem[jj], add=True)
        return 0
    lax.fori_loop(0, n_rounds, _round, 0)
    for jj in range(NBUF):
        pltpu.make_async_copy(
            sbuf.at[jj],
            aggr.at[dst_v.at[(n_rounds - 1) * NBUF + jj]], ssem[jj]).wait()
    plsc.subcore_barrier()

    @pl.when(s < 15)
    def _():
        pltpu.sync_copy(aggr.at[pl.ds(s * 640, 640)], zbuf)
        pltpu.sync_copy(zbuf, part.at[c, pl.ds(s * 640, 640)])

    @pl.when(s == 15)
    def _():
        pltpu.sync_copy(aggr.at[pl.ds(9600, 400)], zbuf.at[pl.ds(0, 400)])
        pltpu.sync_copy(zbuf.at[pl.ds(0, 400)], part.at[c, pl.ds(9600, 400)])


def _sc_aggr1(y1, srcR, dstR, ewR):
    mesh = plsc.VectorSubcoreMesh(core_axis_name="c", subcore_axis_name="s")
    return pl.kernel(
        _sc_aggr1_body,
        out_type=jax.ShapeDtypeStruct((2, N_NODES, D_HID), jnp.float32),
        mesh=mesh,
        compiler_params=pltpu.CompilerParams(use_tc_tiling_on_sc=False, needs_layout_passes=False),
        scratch_types=[
            pltpu.VMEM((ROWS_L1, CHUNK), jnp.int32),
            pltpu.VMEM((ROWS_L1, CHUNK), jnp.int32),
            pltpu.VMEM((ROWS_L1, CHUNK), jnp.float32),
            pltpu.VMEM((NBUF, CHUNK, D_HID), jnp.float32),
            pltpu.VMEM((NBUF, CHUNK, D_HID), jnp.float32),
            pltpu.VMEM((640, D_HID), jnp.float32),
            pltpu.VMEM_SHARED((N_NODES, D_HID), jnp.float32),
            [pltpu.SemaphoreType.DMA] * NBUF,
            [pltpu.SemaphoreType.DMA] * NBUF,
        ],
    )(y1, srcR, dstR, ewR)


def _mid_kernel(p0_ref, p1_ref, r1_ref, b1_ref, wr2_ref, wo2_ref, b2_ref,
                y2_ref, r2pb_ref):
    h = jnp.maximum(
        p0_ref[...] + p1_ref[...] + r1_ref[...] + b1_ref[...], 0.0)
    y2_ref[...] = jnp.sum(h * wr2_ref[...], axis=1, keepdims=True)
    r2pb_ref[...] = (jnp.sum(h * wo2_ref[...], axis=1, keepdims=True)
                     + b2_ref[...])


def _mid(p0, p1, r1, b1, wr2, wo2, b2):
    blk = 1000
    return pl.pallas_call(
        _mid_kernel,
        grid=(N_NODES // blk,),
        in_specs=[
            pl.BlockSpec((blk, D_HID), lambda i: (i, 0)),
            pl.BlockSpec((blk, D_HID), lambda i: (i, 0)),
            pl.BlockSpec((blk, D_HID), lambda i: (i, 0)),
            pl.BlockSpec((1, D_HID), lambda i: (0, 0)),
            pl.BlockSpec((1, D_HID), lambda i: (0, 0)),
            pl.BlockSpec((1, D_HID), lambda i: (0, 0)),
            pl.BlockSpec((1, 1), lambda i: (0, 0)),
        ],
        out_specs=[
            pl.BlockSpec((blk, 1), lambda i: (i, 0)),
            pl.BlockSpec((blk, 1), lambda i: (i, 0)),
        ],
        out_shape=[
            jax.ShapeDtypeStruct((N_NODES, 1), jnp.float32),
            jax.ShapeDtypeStruct((N_NODES, 1), jnp.float32),
        ],
    )(p0, p1, r1, b1, wr2, wo2, b2)


def _sc_aggr2_body(y2, r2pb, srcR, dstR, ewR, out,
                   y2_v, src_v, dst_v, ew_v, pbuf, ibuf, aggr, ssem):
    c = lax.axis_index("c")
    s = lax.axis_index("s")

    @pl.when(c == 0)
    def _():
        pltpu.sync_copy(y2, y2_v)
        pltpu.sync_copy(srcR.at[pl.ds(s * ROWS_L2, ROWS_L2)], src_v)
        pltpu.sync_copy(dstR.at[pl.ds(s * ROWS_L2, ROWS_L2)], dst_v)
        pltpu.sync_copy(ewR.at[pl.ds(s * ROWS_L2, ROWS_L2)], ew_v)

        @pl.when(s < 15)
        def _():
            pltpu.sync_copy(r2pb.at[pl.ds(s * 640, 640)], ibuf)
            pltpu.sync_copy(ibuf, aggr.at[pl.ds(s * 640, 640)])

        @pl.when(s == 15)
        def _():
            pltpu.sync_copy(r2pb.at[pl.ds(9600, 400)], ibuf.at[pl.ds(0, 400)])
            pltpu.sync_copy(ibuf.at[pl.ds(0, 400)], aggr.at[pl.ds(9600, 400)])

        plsc.subcore_barrier()

        n_rounds = ROWS_L2 // NBUF

        def _round(r, _):
            for jj in range(NBUF):
                j = r * NBUF + jj

                @pl.when(r > 0)
                def _():
                    pltpu.make_async_copy(
                        pbuf.at[jj], aggr.at[dst_v.at[j]], ssem[jj]).wait()

                for k in range(CHUNK // 16):
                    idx = src_v[j, pl.ds(16 * k, 16)]
                    vals = plsc.load_gather(y2_v, [idx])
                    w = ew_v[j, pl.ds(16 * k, 16)]
                    pbuf[jj, pl.ds(16 * k, 16)] = vals * w
                pltpu.async_copy(
                    pbuf.at[jj], aggr.at[dst_v.at[j]], ssem[jj], add=True)
            return 0
        lax.fori_loop(0, n_rounds, _round, 0)
        for jj in range(NBUF):
            pltpu.make_async_copy(
                pbuf.at[jj],
                aggr.at[dst_v.at[(n_rounds - 1) * NBUF + jj]],
                ssem[jj]).wait()
        plsc.subcore_barrier()

        @pl.when(s < 15)
        def _():
            pltpu.sync_copy(aggr.at[pl.ds(s * 640, 640)], ibuf)
            pltpu.sync_copy(ibuf, out.at[pl.ds(s * 640, 640)])

        @pl.when(s == 15)
        def _():
            pltpu.sync_copy(aggr.at[pl.ds(9600, 400)], ibuf.at[pl.ds(0, 400)])
            pltpu.sync_copy(ibuf.at[pl.ds(0, 400)], out.at[pl.ds(9600, 400)])


def _sc_aggr2(y2, r2pb, srcR, dstR, ewR):
    mesh = plsc.VectorSubcoreMesh(core_axis_name="c", subcore_axis_name="s")
    return pl.kernel(
        _sc_aggr2_body,
        out_type=jax.ShapeDtypeStruct((N_NODES,), jnp.float32),
        mesh=mesh,
        compiler_params=pltpu.CompilerParams(use_tc_tiling_on_sc=False, needs_layout_passes=False),
        scratch_types=[
            pltpu.VMEM((N_NODES,), jnp.float32),
            pltpu.VMEM((ROWS_L2, CHUNK), jnp.int32),
            pltpu.VMEM((ROWS_L2, CHUNK), jnp.int32),
            pltpu.VMEM((ROWS_L2, CHUNK), jnp.float32),
            pltpu.VMEM((NBUF, CHUNK), jnp.float32),
            pltpu.VMEM((640,), jnp.float32),
            pltpu.VMEM_SHARED((N_NODES,), jnp.float32),
            [pltpu.SemaphoreType.DMA] * NBUF,
        ],
    )(y2, r2pb, srcR, dstR, ewR)


def kernel(x, edge_index, edge_weight, W_rel1, b_rel1, W_root1,
           W_rel2, b_rel2, W_root2):
    npad = EPAD - N_EDGES
    ei = edge_index.astype(jnp.int32)
    srcR = jnp.concatenate(
        [ei[0], jnp.zeros((npad,), jnp.int32)]).reshape(ROWS, CHUNK)
    dstR = jnp.concatenate(
        [ei[1], jnp.zeros((npad,), jnp.int32)]).reshape(ROWS, CHUNK)
    ewR = jnp.concatenate(
        [edge_weight, jnp.zeros((npad,), jnp.float32)]).reshape(ROWS, CHUNK)

    y1, r1 = _proj1(x, W_rel1, W_root1)
    part = _sc_aggr1(y1, srcR, dstR, ewR)
    y2, r2pb = _mid(part[0], part[1], r1, b_rel1.reshape(1, D_HID),
                    W_rel2, W_root2, b_rel2.reshape(1, 1))
    out = _sc_aggr2(y2.reshape(N_NODES), r2pb.reshape(N_NODES),
                    srcR, dstR, ewR)
    return out.reshape(N_NODES, 1)


# speedup vs baseline: 25.1864x; 1.2446x over previous
import functools

import jax
import jax.numpy as jnp
from jax import lax
from jax.experimental import pallas as pl
from jax.experimental.pallas import tpu as pltpu
from jax.experimental.pallas import tpu_sc as plsc

N_NODES = 10000
N_EDGES = 320000
D_IN = 128
D_HID = 16

CHUNK = 128
ROWS = 2560
ROWS_L1 = ROWS // 32
ROWS_L2 = ROWS // 16
EPAD = ROWS * CHUNK
NBUF = 8


def _mm_kernel(x_ref, wr_ref, wo_ref, y1_ref, r1_ref):
    xb = x_ref[...]
    dn = (((1,), (1,)), ((), ()))
    y1_ref[...] = lax.dot_general(xb, wr_ref[...], dn,
                                  preferred_element_type=jnp.float32)
    r1_ref[...] = lax.dot_general(xb, wo_ref[...], dn,
                                  preferred_element_type=jnp.float32)


def _proj1(x, W_rel1, W_root1):
    blk = 1000
    return pl.pallas_call(
        _mm_kernel,
        grid=(N_NODES // blk,),
        in_specs=[
            pl.BlockSpec((blk, D_IN), lambda i: (i, 0)),
            pl.BlockSpec((D_HID, D_IN), lambda i: (0, 0)),
            pl.BlockSpec((D_HID, D_IN), lambda i: (0, 0)),
        ],
        out_specs=[
            pl.BlockSpec((blk, D_HID), lambda i: (i, 0)),
            pl.BlockSpec((blk, D_HID), lambda i: (i, 0)),
        ],
        out_shape=[
            jax.ShapeDtypeStruct((N_NODES, D_HID), jnp.float32),
            jax.ShapeDtypeStruct((N_NODES, D_HID), jnp.float32),
        ],
    )(x, W_rel1, W_root1)


def _sc_aggr1_body(y1, srcR, dstR, ewR, part,
                   src_v, dst_v, ew_v, gbuf, sbuf, zbuf, aggr,
                   gsem, ssem):
    c = lax.axis_index("c")
    s = lax.axis_index("s")
    wid = c * 16 + s
    base = wid * ROWS_L1

    pltpu.sync_copy(srcR.at[pl.ds(base, ROWS_L1)], src_v)
    pltpu.sync_copy(dstR.at[pl.ds(base, ROWS_L1)], dst_v)
    pltpu.sync_copy(ewR.at[pl.ds(base, ROWS_L1)], ew_v)

    def _z(i, _):
        zbuf[i, :] = jnp.zeros((16,), jnp.float32)
        return 0
    lax.fori_loop(0, 640, _z, 0)

    @pl.when(s < 15)
    def _():
        pltpu.sync_copy(zbuf, aggr.at[pl.ds(s * 640, 640)])

    @pl.when(s == 15)
    def _():
        pltpu.sync_copy(zbuf.at[pl.ds(0, 400)], aggr.at[pl.ds(9600, 400)])

    plsc.subcore_barrier()

    n_rounds = ROWS_L1 // NBUF
    for jj in range(NBUF):
        pltpu.async_copy(y1.at[src_v.at[jj]], gbuf.at[jj], gsem[jj])

    def _round(r, _):
        for jj in range(NBUF):
            j = r * NBUF + jj
            pltpu.make_async_copy(
                y1.at[src_v.at[j]], gbuf.at[jj], gsem[jj]).wait()

            @pl.when(r > 0)
            def _():
                pltpu.make_async_copy(
                    sbuf.at[jj], aggr.at[dst_v.at[j]], ssem[jj]).wait()

            for k in range(CHUNK // 16):
                w = ew_v[j, pl.ds(16 * k, 16)]
                for e in range(16):
                    sbuf[jj, 16 * k + e, :] = gbuf[jj, 16 * k + e, :] * w[e]

            @pl.when(r < n_rounds - 1)
            def _():
                pltpu.async_copy(
                    y1.at[src_v.at[j + NBUF]], gbuf.at[jj], gsem[jj])

            pltpu.async_copy(
                sbuf.at[jj], aggr.at[dst_v.at[j]], ssem[jj], add=True)
        return 0
    lax.fori_loop(0, n_rounds, _round, 0)
    for jj in range(NBUF):
        pltpu.make_async_copy(
            sbuf.at[jj],
            aggr.at[dst_v.at[(n_rounds - 1) * NBUF + jj]], ssem[jj]).wait()
    plsc.subcore_barrier()

    @pl.when(s < 15)
    def _():
        pltpu.sync_copy(aggr.at[pl.ds(s * 640, 640)], zbuf)
        pltpu.sync_copy(zbuf, part.at[c, pl.ds(s * 640, 640)])

    @pl.when(s == 15)
    def _():
        pltpu.sync_copy(aggr.at[pl.ds(9600, 400)], zbuf.at[pl.ds(0, 400)])
        pltpu.sync_copy(zbuf.at[pl.ds(0, 400)], part.at[c, pl.ds(9600, 400)])


def _sc_aggr1(y1, srcR, dstR, ewR):
    mesh = plsc.VectorSubcoreMesh(core_axis_name="c", subcore_axis_name="s")
    return pl.kernel(
        _sc_aggr1_body,
        out_type=jax.ShapeDtypeStruct((2, N_NODES, D_HID), jnp.float32),
        mesh=mesh,
        compiler_params=pltpu.CompilerParams(use_tc_tiling_on_sc=False, needs_layout_passes=False),
        scratch_types=[
            pltpu.VMEM((ROWS_L1, CHUNK), jnp.int32),
            pltpu.VMEM((ROWS_L1, CHUNK), jnp.int32),
            pltpu.VMEM((ROWS_L1, CHUNK), jnp.float32),
            pltpu.VMEM((NBUF, CHUNK, D_HID), jnp.float32),
            pltpu.VMEM((NBUF, CHUNK, D_HID), jnp.float32),
            pltpu.VMEM((640, D_HID), jnp.float32),
            pltpu.VMEM_SHARED((N_NODES, D_HID), jnp.float32),
            [pltpu.SemaphoreType.DMA] * NBUF,
            [pltpu.SemaphoreType.DMA] * NBUF,
        ],
    )(y1, srcR, dstR, ewR)


def _sc_aggr2_body(part, r1, b1, wr2, wo2, b2, srcR, dstR, ewR, out,
                   y2_v, src_v, dst_v, ew_v, p0_v, p1_v, r1_v, ht,
                   cst_v, y2loc, r2loc, pbuf, ibuf, y2sh, aggr,
                   esem, ssem):
    c = lax.axis_index("c")
    s = lax.axis_index("s")

    @pl.when(c == 0)
    def _():
        pltpu.async_copy(srcR.at[pl.ds(s * ROWS_L2, ROWS_L2)], src_v, esem)
        pltpu.async_copy(dstR.at[pl.ds(s * ROWS_L2, ROWS_L2)], dst_v, esem)
        pltpu.async_copy(ewR.at[pl.ds(s * ROWS_L2, ROWS_L2)], ew_v, esem)

        pltpu.sync_copy(b1, cst_v.at[0])
        pltpu.sync_copy(wr2, cst_v.at[1])
        pltpu.sync_copy(wo2, cst_v.at[2])
        pltpu.sync_copy(b2, cst_v.at[3])
        b1v = cst_v[0, :]
        wr2v = cst_v[1, :]
        wo2v = cst_v[2, :]
        b2s = cst_v[3, :][0]

        base = s * 640
        nn_s = [640, 400]

        def _dense(nn):
            pltpu.sync_copy(part.at[0, pl.ds(base, nn)],
                            p0_v.at[pl.ds(0, nn)])
            pltpu.sync_copy(part.at[1, pl.ds(base, nn)],
                            p1_v.at[pl.ds(0, nn)])
            pltpu.sync_copy(r1.at[pl.ds(base, nn)], r1_v.at[pl.ds(0, nn)])
            lanes = lax.broadcasted_iota(jnp.int32, (16,), 0)

            def _grp(g, _):
                for n in range(16):
                    h = jnp.maximum(
                        p0_v[16 * g + n, :] + p1_v[16 * g + n, :]
                        + r1_v[16 * g + n, :] + b1v, 0.0)
                    plsc.store_scatter(
                        ht, [lanes, jnp.full((16,), n, jnp.int32)], h)
                acc = jnp.zeros((16,), jnp.float32)
                acc2 = jnp.zeros((16,), jnp.float32)
                for f in range(16):
                    col = ht[f, :]
                    acc = acc + col * wr2v[f]
                    acc2 = acc2 + col * wo2v[f]
                y2loc[pl.ds(16 * g, 16)] = acc
                r2loc[pl.ds(16 * g, 16)] = acc2 + b2s
                return 0
            lax.fori_loop(0, nn // 16, _grp, 0)
            pltpu.sync_copy(y2loc.at[pl.ds(0, nn)],
                            y2sh.at[pl.ds(base, nn)])
            pltpu.sync_copy(r2loc.at[pl.ds(0, nn)],
                            aggr.at[pl.ds(base, nn)])

        @pl.when(s < 15)
        def _():
            _dense(nn_s[0])

        @pl.when(s == 15)
        def _():
            _dense(nn_s[1])

        plsc.subcore_barrier()
        pltpu.sync_copy(y2sh, y2_v)

        pltpu.make_async_copy(
            srcR.at[pl.ds(s * ROWS_L2, ROWS_L2)], src_v, esem).wait()
        pltpu.make_async_copy(
            dstR.at[pl.ds(s * ROWS_L2, ROWS_L2)], dst_v, esem).wait()
        pltpu.make_async_copy(
            ewR.at[pl.ds(s * ROWS_L2, ROWS_L2)], ew_v, esem).wait()

        n_rounds = ROWS_L2 // NBUF

        def _round(r, _):
            for jj in range(NBUF):
                j = r * NBUF + jj

                @pl.when(r > 0)
                def _():
                    pltpu.make_async_copy(
                        pbuf.at[jj], aggr.at[dst_v.at[j]], ssem[jj]).wait()

                for k in range(CHUNK // 16):
                    idx = src_v[j, pl.ds(16 * k, 16)]
                    vals = plsc.load_gather(y2_v, [idx])
                    w = ew_v[j, pl.ds(16 * k, 16)]
                    pbuf[jj, pl.ds(16 * k, 16)] = vals * w
                pltpu.async_copy(
                    pbuf.at[jj], aggr.at[dst_v.at[j]], ssem[jj], add=True)
            return 0
        lax.fori_loop(0, n_rounds, _round, 0)
        for jj in range(NBUF):
            pltpu.make_async_copy(
                pbuf.at[jj],
                aggr.at[dst_v.at[(n_rounds - 1) * NBUF + jj]],
                ssem[jj]).wait()
        plsc.subcore_barrier()

        @pl.when(s < 15)
        def _():
            pltpu.sync_copy(aggr.at[pl.ds(s * 640, 640)], ibuf)
            pltpu.sync_copy(ibuf, out.at[pl.ds(s * 640, 640)])

        @pl.when(s == 15)
        def _():
            pltpu.sync_copy(aggr.at[pl.ds(9600, 400)], ibuf.at[pl.ds(0, 400)])
            pltpu.sync_copy(ibuf.at[pl.ds(0, 400)], out.at[pl.ds(9600, 400)])


def _sc_aggr2(part, r1, b1, wr2, wo2, b2, srcR, dstR, ewR):
    mesh = plsc.VectorSubcoreMesh(core_axis_name="c", subcore_axis_name="s")
    return pl.kernel(
        _sc_aggr2_body,
        out_type=jax.ShapeDtypeStruct((N_NODES,), jnp.float32),
        mesh=mesh,
        compiler_params=pltpu.CompilerParams(use_tc_tiling_on_sc=False, needs_layout_passes=False),
        scratch_types=[
            pltpu.VMEM((N_NODES,), jnp.float32),
            pltpu.VMEM((ROWS_L2, CHUNK), jnp.int32),
            pltpu.VMEM((ROWS_L2, CHUNK), jnp.int32),
            pltpu.VMEM((ROWS_L2, CHUNK), jnp.float32),
            pltpu.VMEM((640, D_HID), jnp.float32),
            pltpu.VMEM((640, D_HID), jnp.float32),
            pltpu.VMEM((640, D_HID), jnp.float32),
            pltpu.VMEM((16, 16), jnp.float32),
            pltpu.VMEM((4, 16), jnp.float32),
            pltpu.VMEM((640,), jnp.float32),
            pltpu.VMEM((640,), jnp.float32),
            pltpu.VMEM((NBUF, CHUNK), jnp.float32),
            pltpu.VMEM((640,), jnp.float32),
            pltpu.VMEM_SHARED((N_NODES,), jnp.float32),
            pltpu.VMEM_SHARED((N_NODES,), jnp.float32),
            pltpu.SemaphoreType.DMA,
            [pltpu.SemaphoreType.DMA] * NBUF,
        ],
    )(part, r1, b1, wr2, wo2, b2, srcR, dstR, ewR)


def kernel(x, edge_index, edge_weight, W_rel1, b_rel1, W_root1,
           W_rel2, b_rel2, W_root2):
    npad = EPAD - N_EDGES
    ei = edge_index.astype(jnp.int32)
    srcR = jnp.concatenate(
        [ei[0], jnp.zeros((npad,), jnp.int32)]).reshape(ROWS, CHUNK)
    dstR = jnp.concatenate(
        [ei[1], jnp.zeros((npad,), jnp.int32)]).reshape(ROWS, CHUNK)
    ewR = jnp.concatenate(
        [edge_weight, jnp.zeros((npad,), jnp.float32)]).reshape(ROWS, CHUNK)

    y1, r1 = _proj1(x, W_rel1, W_root1)
    part = _sc_aggr1(y1, srcR, dstR, ewR)
    out = _sc_aggr2(part, r1, b_rel1,
                    W_rel2.reshape(D_HID), W_root2.reshape(D_HID),
                    jnp.broadcast_to(b_rel2, (D_HID,)),
                    srcR, dstR, ewR)
    return out.reshape(N_NODES, 1)


# speedup vs baseline: 28.5401x; 1.1332x over previous
import functools

import jax
import jax.numpy as jnp
from jax import lax
from jax.experimental import pallas as pl
from jax.experimental.pallas import tpu as pltpu
from jax.experimental.pallas import tpu_sc as plsc

N_NODES = 10000
N_EDGES = 320000
D_IN = 128
D_HID = 16

CHUNK = 128
ROWS = 2560
ROWS_L1 = ROWS // 32
ROWS_L2 = ROWS // 16
EPAD = ROWS * CHUNK
NBUF = 8


def _mm_kernel(x_ref, wr_ref, wo_ref, y1_ref, r1_ref):
    xb = x_ref[...]
    dn = (((1,), (1,)), ((), ()))
    y1_ref[...] = lax.dot_general(xb, wr_ref[...], dn,
                                  preferred_element_type=jnp.float32)
    r1_ref[...] = lax.dot_general(xb, wo_ref[...], dn,
                                  preferred_element_type=jnp.float32)


def _proj1(x, W_rel1, W_root1):
    blk = 1000
    return pl.pallas_call(
        _mm_kernel,
        grid=(N_NODES // blk,),
        in_specs=[
            pl.BlockSpec((blk, D_IN), lambda i: (i, 0)),
            pl.BlockSpec((D_HID, D_IN), lambda i: (0, 0)),
            pl.BlockSpec((D_HID, D_IN), lambda i: (0, 0)),
        ],
        out_specs=[
            pl.BlockSpec((blk, D_HID), lambda i: (i, 0)),
            pl.BlockSpec((blk, D_HID), lambda i: (i, 0)),
        ],
        out_shape=[
            jax.ShapeDtypeStruct((N_NODES, D_HID), jnp.float32),
            jax.ShapeDtypeStruct((N_NODES, D_HID), jnp.float32),
        ],
    )(x, W_rel1, W_root1)


def _sc_aggr1_body(y1, eiR, ewR, part,
                   src_v, dst_v, ew_v, gbuf, sbuf, zbuf, aggr,
                   gsem, ssem):
    c = lax.axis_index("c")
    s = lax.axis_index("s")
    wid = c * 16 + s
    base = wid * ROWS_L1

    pltpu.sync_copy(eiR.at[0, pl.ds(base, ROWS_L1)], src_v)
    pltpu.sync_copy(eiR.at[1, pl.ds(base, ROWS_L1)], dst_v)
    pltpu.sync_copy(ewR.at[pl.ds(base, ROWS_L1)], ew_v)

    def _z(i, _):
        zbuf[i, :] = jnp.zeros((16,), jnp.float32)
        return 0
    lax.fori_loop(0, 640, _z, 0)

    @pl.when(s < 15)
    def _():
        pltpu.sync_copy(zbuf, aggr.at[pl.ds(s * 640, 640)])

    @pl.when(s == 15)
    def _():
        pltpu.sync_copy(zbuf.at[pl.ds(0, 400)], aggr.at[pl.ds(9600, 400)])

    plsc.subcore_barrier()

    n_rounds = ROWS_L1 // NBUF
    for jj in range(NBUF):
        pltpu.async_copy(y1.at[src_v.at[jj]], gbuf.at[jj], gsem[jj])

    def _round(r, _):
        for jj in range(NBUF):
            j = r * NBUF + jj
            pltpu.make_async_copy(
                y1.at[src_v.at[j]], gbuf.at[jj], gsem[jj]).wait()

            @pl.when(r > 0)
            def _():
                pltpu.make_async_copy(
                    sbuf.at[jj], aggr.at[dst_v.at[j]], ssem[jj]).wait()

            for k in range(CHUNK // 16):
                w = ew_v[j, pl.ds(16 * k, 16)]
                for e in range(16):
                    sbuf[jj, 16 * k + e, :] = gbuf[jj, 16 * k + e, :] * w[e]

            @pl.when(r < n_rounds - 1)
            def _():
                pltpu.async_copy(
                    y1.at[src_v.at[j + NBUF]], gbuf.at[jj], gsem[jj])

            pltpu.async_copy(
                sbuf.at[jj], aggr.at[dst_v.at[j]], ssem[jj], add=True)
        return 0
    lax.fori_loop(0, n_rounds, _round, 0)
    for jj in range(NBUF):
        pltpu.make_async_copy(
            sbuf.at[jj],
            aggr.at[dst_v.at[(n_rounds - 1) * NBUF + jj]], ssem[jj]).wait()
    plsc.subcore_barrier()

    @pl.when(s < 15)
    def _():
        pltpu.sync_copy(aggr.at[pl.ds(s * 640, 640)], zbuf)
        pltpu.sync_copy(zbuf, part.at[c, pl.ds(s * 640, 640)])

    @pl.when(s == 15)
    def _():
        pltpu.sync_copy(aggr.at[pl.ds(9600, 400)], zbuf.at[pl.ds(0, 400)])
        pltpu.sync_copy(zbuf.at[pl.ds(0, 400)], part.at[c, pl.ds(9600, 400)])


def _sc_aggr1(y1, eiR, ewR):
    mesh = plsc.VectorSubcoreMesh(core_axis_name="c", subcore_axis_name="s")
    return pl.kernel(
        _sc_aggr1_body,
        out_type=jax.ShapeDtypeStruct((2, N_NODES, D_HID), jnp.float32),
        mesh=mesh,
        compiler_params=pltpu.CompilerParams(use_tc_tiling_on_sc=False, needs_layout_passes=False),
        scratch_types=[
            pltpu.VMEM((ROWS_L1, CHUNK), jnp.int32),
            pltpu.VMEM((ROWS_L1, CHUNK), jnp.int32),
            pltpu.VMEM((ROWS_L1, CHUNK), jnp.float32),
            pltpu.VMEM((NBUF, CHUNK, D_HID), jnp.float32),
            pltpu.VMEM((NBUF, CHUNK, D_HID), jnp.float32),
            pltpu.VMEM((640, D_HID), jnp.float32),
            pltpu.VMEM_SHARED((N_NODES, D_HID), jnp.float32),
            [pltpu.SemaphoreType.DMA] * NBUF,
            [pltpu.SemaphoreType.DMA] * NBUF,
        ],
    )(y1, eiR, ewR)


def _sc_aggr2_body(part, r1, b1, wr2, wo2, b2, eiR, ewR, out,
                   y2_v, src_v, dst_v, ew_v, p0_v, p1_v, r1_v, ht,
                   cst_v, y2loc, r2loc, pbuf, ibuf, y2sh, aggr,
                   esem, ssem):
    c = lax.axis_index("c")
    s = lax.axis_index("s")

    @pl.when(c == 0)
    def _():
        pltpu.async_copy(eiR.at[0, pl.ds(s * ROWS_L2, ROWS_L2)], src_v, esem)
        pltpu.async_copy(eiR.at[1, pl.ds(s * ROWS_L2, ROWS_L2)], dst_v, esem)
        pltpu.async_copy(ewR.at[pl.ds(s * ROWS_L2, ROWS_L2)], ew_v, esem)

        pltpu.sync_copy(b1, cst_v.at[0])
        pltpu.sync_copy(wr2, cst_v.at[1])
        pltpu.sync_copy(wo2, cst_v.at[2])
        pltpu.sync_copy(b2, cst_v.at[3])
        b1v = cst_v[0, :]
        wr2v = cst_v[1, :]
        wo2v = cst_v[2, :]
        b2s = cst_v[3, :][0]

        base = s * 640
        nn_s = [640, 400]

        def _dense(nn):
            pltpu.sync_copy(part.at[0, pl.ds(base, nn)],
                            p0_v.at[pl.ds(0, nn)])
            pltpu.sync_copy(part.at[1, pl.ds(base, nn)],
                            p1_v.at[pl.ds(0, nn)])
            pltpu.sync_copy(r1.at[pl.ds(base, nn)], r1_v.at[pl.ds(0, nn)])
            lanes = lax.broadcasted_iota(jnp.int32, (16,), 0)

            def _grp(g, _):
                for n in range(16):
                    h = jnp.maximum(
                        p0_v[16 * g + n, :] + p1_v[16 * g + n, :]
                        + r1_v[16 * g + n, :] + b1v, 0.0)
                    plsc.store_scatter(
                        ht, [lanes, jnp.full((16,), n, jnp.int32)], h)
                acc = jnp.zeros((16,), jnp.float32)
                acc2 = jnp.zeros((16,), jnp.float32)
                for f in range(16):
                    col = ht[f, :]
                    acc = acc + col * wr2v[f]
                    acc2 = acc2 + col * wo2v[f]
                y2loc[pl.ds(16 * g, 16)] = acc
                r2loc[pl.ds(16 * g, 16)] = acc2 + b2s
                return 0
            lax.fori_loop(0, nn // 16, _grp, 0)
            pltpu.sync_copy(y2loc.at[pl.ds(0, nn)],
                            y2sh.at[pl.ds(base, nn)])
            pltpu.sync_copy(r2loc.at[pl.ds(0, nn)],
                            aggr.at[pl.ds(base, nn)])

        @pl.when(s < 15)
        def _():
            _dense(nn_s[0])

        @pl.when(s == 15)
        def _():
            _dense(nn_s[1])

        plsc.subcore_barrier()
        pltpu.sync_copy(y2sh, y2_v)

        pltpu.make_async_copy(
            eiR.at[0, pl.ds(s * ROWS_L2, ROWS_L2)], src_v, esem).wait()
        pltpu.make_async_copy(
            eiR.at[1, pl.ds(s * ROWS_L2, ROWS_L2)], dst_v, esem).wait()
        pltpu.make_async_copy(
            ewR.at[pl.ds(s * ROWS_L2, ROWS_L2)], ew_v, esem).wait()

        n_rounds = ROWS_L2 // NBUF

        def _round(r, _):
            for jj in range(NBUF):
                j = r * NBUF + jj

                @pl.when(r > 0)
                def _():
                    pltpu.make_async_copy(
                        pbuf.at[jj], aggr.at[dst_v.at[j]], ssem[jj]).wait()

                for k in range(CHUNK // 16):
                    idx = src_v[j, pl.ds(16 * k, 16)]
                    vals = plsc.load_gather(y2_v, [idx])
                    w = ew_v[j, pl.ds(16 * k, 16)]
                    pbuf[jj, pl.ds(16 * k, 16)] = vals * w
                pltpu.async_copy(
                    pbuf.at[jj], aggr.at[dst_v.at[j]], ssem[jj], add=True)
            return 0
        lax.fori_loop(0, n_rounds, _round, 0)
        for jj in range(NBUF):
            pltpu.make_async_copy(
                pbuf.at[jj],
                aggr.at[dst_v.at[(n_rounds - 1) * NBUF + jj]],
                ssem[jj]).wait()
        plsc.subcore_barrier()

        @pl.when(s < 15)
        def _():
            pltpu.sync_copy(aggr.at[pl.ds(s * 640, 640)], ibuf)
            pltpu.sync_copy(ibuf, out.at[pl.ds(s * 640, 640)])

        @pl.when(s == 15)
        def _():
            pltpu.sync_copy(aggr.at[pl.ds(9600, 400)], ibuf.at[pl.ds(0, 400)])
            pltpu.sync_copy(ibuf.at[pl.ds(0, 400)], out.at[pl.ds(9600, 400)])


def _sc_aggr2(part, r1, b1, wr2, wo2, b2, eiR, ewR):
    mesh = plsc.VectorSubcoreMesh(core_axis_name="c", subcore_axis_name="s")
    return pl.kernel(
        _sc_aggr2_body,
        out_type=jax.ShapeDtypeStruct((N_NODES,), jnp.float32),
        mesh=mesh,
        compiler_params=pltpu.CompilerParams(use_tc_tiling_on_sc=False, needs_layout_passes=False),
        scratch_types=[
            pltpu.VMEM((N_NODES,), jnp.float32),
            pltpu.VMEM((ROWS_L2, CHUNK), jnp.int32),
            pltpu.VMEM((ROWS_L2, CHUNK), jnp.int32),
            pltpu.VMEM((ROWS_L2, CHUNK), jnp.float32),
            pltpu.VMEM((640, D_HID), jnp.float32),
            pltpu.VMEM((640, D_HID), jnp.float32),
            pltpu.VMEM((640, D_HID), jnp.float32),
            pltpu.VMEM((16, 16), jnp.float32),
            pltpu.VMEM((4, 16), jnp.float32),
            pltpu.VMEM((640,), jnp.float32),
            pltpu.VMEM((640,), jnp.float32),
            pltpu.VMEM((NBUF, CHUNK), jnp.float32),
            pltpu.VMEM((640,), jnp.float32),
            pltpu.VMEM_SHARED((N_NODES,), jnp.float32),
            pltpu.VMEM_SHARED((N_NODES,), jnp.float32),
            pltpu.SemaphoreType.DMA,
            [pltpu.SemaphoreType.DMA] * NBUF,
        ],
    )(part, r1, b1, wr2, wo2, b2, eiR, ewR)


def kernel(x, edge_index, edge_weight, W_rel1, b_rel1, W_root1,
           W_rel2, b_rel2, W_root2):
    npad = EPAD - N_EDGES
    ei = edge_index.astype(jnp.int32)
    eiR = jnp.concatenate(
        [ei, jnp.zeros((2, npad), jnp.int32)], axis=1).reshape(
            2, ROWS, CHUNK)
    ewR = jnp.concatenate(
        [edge_weight, jnp.zeros((npad,), jnp.float32)]).reshape(ROWS, CHUNK)

    y1, r1 = _proj1(x, W_rel1, W_root1)
    part = _sc_aggr1(y1, eiR, ewR)
    out = _sc_aggr2(part, r1, b_rel1,
                    W_rel2.reshape(D_HID), W_root2.reshape(D_HID),
                    jnp.broadcast_to(b_rel2, (D_HID,)),
                    eiR, ewR)
    return out.reshape(N_NODES, 1)


# speedup vs baseline: 29.4399x; 1.0315x over previous
import functools

import jax
import jax.numpy as jnp
from jax import lax
from jax.experimental import pallas as pl
from jax.experimental.pallas import tpu as pltpu
from jax.experimental.pallas import tpu_sc as plsc

N_NODES = 10000
N_EDGES = 320000
D_IN = 128
D_HID = 16

CHUNK = 128
ROWS = 2560
ROWS_L1 = ROWS // 32
ROWS_L2 = ROWS // 16
EPAD = ROWS * CHUNK
NBUF = 8


def _mm_kernel(x_ref, wr_ref, wo_ref, y1_ref, r1_ref):
    xb = x_ref[...]
    dn = (((1,), (1,)), ((), ()))
    y1_ref[...] = lax.dot_general(xb, wr_ref[...], dn,
                                  preferred_element_type=jnp.float32)
    r1_ref[...] = lax.dot_general(xb, wo_ref[...], dn,
                                  preferred_element_type=jnp.float32)


def _proj1(x, W_rel1, W_root1):
    blk = 1000
    return pl.pallas_call(
        _mm_kernel,
        grid=(N_NODES // blk,),
        in_specs=[
            pl.BlockSpec((blk, D_IN), lambda i: (i, 0)),
            pl.BlockSpec((D_HID, D_IN), lambda i: (0, 0)),
            pl.BlockSpec((D_HID, D_IN), lambda i: (0, 0)),
        ],
        out_specs=[
            pl.BlockSpec((blk, D_HID), lambda i: (i, 0)),
            pl.BlockSpec((blk, D_HID), lambda i: (i, 0)),
        ],
        out_shape=[
            jax.ShapeDtypeStruct((N_NODES, D_HID), jnp.float32),
            jax.ShapeDtypeStruct((N_NODES, D_HID), jnp.float32),
        ],
    )(x, W_rel1, W_root1)


def _sc_aggr1_body(y1, eiR, ewR, part,
                   src_v, dst_v, ew_v, gbuf, sbuf, zbuf, aggr,
                   gsem, ssem):
    c = lax.axis_index("c")
    s = lax.axis_index("s")
    wid = c * 16 + s
    base = wid * ROWS_L1

    pltpu.async_copy(eiR.at[0, pl.ds(base, ROWS_L1)], src_v, gsem[0])
    pltpu.async_copy(eiR.at[1, pl.ds(base, ROWS_L1)], dst_v, gsem[1])
    pltpu.async_copy(ewR.at[pl.ds(base, ROWS_L1)], ew_v, gsem[2])

    def _z(i, _):
        zbuf[i, :] = jnp.zeros((16,), jnp.float32)
        return 0
    lax.fori_loop(0, 640, _z, 0)

    @pl.when(s < 15)
    def _():
        pltpu.sync_copy(zbuf, aggr.at[pl.ds(s * 640, 640)])

    @pl.when(s == 15)
    def _():
        pltpu.sync_copy(zbuf.at[pl.ds(0, 400)], aggr.at[pl.ds(9600, 400)])

    pltpu.make_async_copy(
        eiR.at[0, pl.ds(base, ROWS_L1)], src_v, gsem[0]).wait()
    pltpu.make_async_copy(
        eiR.at[1, pl.ds(base, ROWS_L1)], dst_v, gsem[1]).wait()
    pltpu.make_async_copy(
        ewR.at[pl.ds(base, ROWS_L1)], ew_v, gsem[2]).wait()
    plsc.subcore_barrier()

    n_rounds = ROWS_L1 // NBUF
    for jj in range(NBUF):
        pltpu.async_copy(y1.at[src_v.at[jj]], gbuf.at[jj], gsem[jj])

    def _round(r, _):
        for jj in range(NBUF):
            j = r * NBUF + jj
            pltpu.make_async_copy(
                y1.at[src_v.at[j]], gbuf.at[jj], gsem[jj]).wait()

            @pl.when(r > 0)
            def _():
                pltpu.make_async_copy(
                    sbuf.at[jj], aggr.at[dst_v.at[j]], ssem[jj]).wait()

            for k in range(CHUNK // 16):
                w = ew_v[j, pl.ds(16 * k, 16)]
                for e in range(16):
                    sbuf[jj, 16 * k + e, :] = gbuf[jj, 16 * k + e, :] * w[e]

            @pl.when(r < n_rounds - 1)
            def _():
                pltpu.async_copy(
                    y1.at[src_v.at[j + NBUF]], gbuf.at[jj], gsem[jj])

            pltpu.async_copy(
                sbuf.at[jj], aggr.at[dst_v.at[j]], ssem[jj], add=True)
        return 0
    lax.fori_loop(0, n_rounds, _round, 0)
    for jj in range(NBUF):
        pltpu.make_async_copy(
            sbuf.at[jj],
            aggr.at[dst_v.at[(n_rounds - 1) * NBUF + jj]], ssem[jj]).wait()
    plsc.subcore_barrier()

    @pl.when(s < 15)
    def _():
        pltpu.sync_copy(aggr.at[pl.ds(s * 640, 640)],
                        part.at[c, pl.ds(s * 640, 640)])

    @pl.when(s == 15)
    def _():
        pltpu.sync_copy(aggr.at[pl.ds(9600, 400)],
                        part.at[c, pl.ds(9600, 400)])


def _sc_aggr1(y1, eiR, ewR):
    mesh = plsc.VectorSubcoreMesh(core_axis_name="c", subcore_axis_name="s")
    return pl.kernel(
        _sc_aggr1_body,
        out_type=jax.ShapeDtypeStruct((2, N_NODES, D_HID), jnp.float32),
        mesh=mesh,
        compiler_params=pltpu.CompilerParams(use_tc_tiling_on_sc=False, needs_layout_passes=False),
        scratch_types=[
            pltpu.VMEM((ROWS_L1, CHUNK), jnp.int32),
            pltpu.VMEM((ROWS_L1, CHUNK), jnp.int32),
            pltpu.VMEM((ROWS_L1, CHUNK), jnp.float32),
            pltpu.VMEM((NBUF, CHUNK, D_HID), jnp.float32),
            pltpu.VMEM((NBUF, CHUNK, D_HID), jnp.float32),
            pltpu.VMEM((640, D_HID), jnp.float32),
            pltpu.VMEM_SHARED((N_NODES, D_HID), jnp.float32),
            [pltpu.SemaphoreType.DMA] * NBUF,
            [pltpu.SemaphoreType.DMA] * NBUF,
        ],
    )(y1, eiR, ewR)


def _sc_aggr2_body(part, r1, b1, wr2, wo2, b2, eiR, ewR, out,
                   y2_v, src_v, dst_v, ew_v, p0_v, p1_v, r1_v, ht,
                   cst_v, y2loc, r2loc, pbuf, ibuf, y2sh, aggr,
                   esem, ssem):
    c = lax.axis_index("c")
    s = lax.axis_index("s")

    @pl.when(c == 0)
    def _():
        pltpu.async_copy(eiR.at[0, pl.ds(s * ROWS_L2, ROWS_L2)], src_v, esem)
        pltpu.async_copy(eiR.at[1, pl.ds(s * ROWS_L2, ROWS_L2)], dst_v, esem)
        pltpu.async_copy(ewR.at[pl.ds(s * ROWS_L2, ROWS_L2)], ew_v, esem)

        pltpu.async_copy(b1, cst_v.at[0], ssem[0])
        pltpu.async_copy(wr2, cst_v.at[1], ssem[1])
        pltpu.async_copy(wo2, cst_v.at[2], ssem[2])
        pltpu.async_copy(b2, cst_v.at[3], ssem[3])
        pltpu.make_async_copy(b1, cst_v.at[0], ssem[0]).wait()
        pltpu.make_async_copy(wr2, cst_v.at[1], ssem[1]).wait()
        pltpu.make_async_copy(wo2, cst_v.at[2], ssem[2]).wait()
        pltpu.make_async_copy(b2, cst_v.at[3], ssem[3]).wait()
        b1v = cst_v[0, :]
        wr2v = cst_v[1, :]
        wo2v = cst_v[2, :]
        b2s = cst_v[3, :][0]

        base = s * 640
        nn_s = [640, 400]

        def _dense(nn):
            pltpu.async_copy(part.at[0, pl.ds(base, nn)],
                             p0_v.at[pl.ds(0, nn)], ssem[4])
            pltpu.async_copy(part.at[1, pl.ds(base, nn)],
                             p1_v.at[pl.ds(0, nn)], ssem[5])
            pltpu.async_copy(r1.at[pl.ds(base, nn)],
                             r1_v.at[pl.ds(0, nn)], ssem[6])
            pltpu.make_async_copy(part.at[0, pl.ds(base, nn)],
                                  p0_v.at[pl.ds(0, nn)], ssem[4]).wait()
            pltpu.make_async_copy(part.at[1, pl.ds(base, nn)],
                                  p1_v.at[pl.ds(0, nn)], ssem[5]).wait()
            pltpu.make_async_copy(r1.at[pl.ds(base, nn)],
                                  r1_v.at[pl.ds(0, nn)], ssem[6]).wait()
            lanes = lax.broadcasted_iota(jnp.int32, (16,), 0)

            def _grp(g, _):
                for n in range(16):
                    h = jnp.maximum(
                        p0_v[16 * g + n, :] + p1_v[16 * g + n, :]
                        + r1_v[16 * g + n, :] + b1v, 0.0)
                    plsc.store_scatter(
                        ht, [lanes, jnp.full((16,), n, jnp.int32)], h)
                acc = jnp.zeros((16,), jnp.float32)
                acc2 = jnp.zeros((16,), jnp.float32)
                for f in range(16):
                    col = ht[f, :]
                    acc = acc + col * wr2v[f]
                    acc2 = acc2 + col * wo2v[f]
                y2loc[pl.ds(16 * g, 16)] = acc
                r2loc[pl.ds(16 * g, 16)] = acc2 + b2s
                return 0
            lax.fori_loop(0, nn // 16, _grp, 0)
            pltpu.sync_copy(y2loc.at[pl.ds(0, nn)],
                            y2sh.at[pl.ds(base, nn)])
            pltpu.sync_copy(r2loc.at[pl.ds(0, nn)],
                            aggr.at[pl.ds(base, nn)])

        @pl.when(s < 15)
        def _():
            _dense(nn_s[0])

        @pl.when(s == 15)
        def _():
            _dense(nn_s[1])

        plsc.subcore_barrier()
        pltpu.sync_copy(y2sh, y2_v)

        pltpu.make_async_copy(
            eiR.at[0, pl.ds(s * ROWS_L2, ROWS_L2)], src_v, esem).wait()
        pltpu.make_async_copy(
            eiR.at[1, pl.ds(s * ROWS_L2, ROWS_L2)], dst_v, esem).wait()
        pltpu.make_async_copy(
            ewR.at[pl.ds(s * ROWS_L2, ROWS_L2)], ew_v, esem).wait()

        n_rounds = ROWS_L2 // NBUF

        def _round(r, _):
            for jj in range(NBUF):
                j = r * NBUF + jj

                @pl.when(r > 0)
                def _():
                    pltpu.make_async_copy(
                        pbuf.at[jj], aggr.at[dst_v.at[j]], ssem[jj]).wait()

                for k in range(CHUNK // 16):
                    idx = src_v[j, pl.ds(16 * k, 16)]
                    vals = plsc.load_gather(y2_v, [idx])
                    w = ew_v[j, pl.ds(16 * k, 16)]
                    pbuf[jj, pl.ds(16 * k, 16)] = vals * w
                pltpu.async_copy(
                    pbuf.at[jj], aggr.at[dst_v.at[j]], ssem[jj], add=True)
            return 0
        lax.fori_loop(0, n_rounds, _round, 0)
        for jj in range(NBUF):
            pltpu.make_async_copy(
                pbuf.at[jj],
                aggr.at[dst_v.at[(n_rounds - 1) * NBUF + jj]],
                ssem[jj]).wait()
        plsc.subcore_barrier()

        @pl.when(s < 15)
        def _():
            pltpu.sync_copy(aggr.at[pl.ds(s * 640, 640)],
                            out.at[pl.ds(s * 640, 640)])

        @pl.when(s == 15)
        def _():
            pltpu.sync_copy(aggr.at[pl.ds(9600, 400)],
                            out.at[pl.ds(9600, 400)])


def _sc_aggr2(part, r1, b1, wr2, wo2, b2, eiR, ewR):
    mesh = plsc.VectorSubcoreMesh(core_axis_name="c", subcore_axis_name="s")
    return pl.kernel(
        _sc_aggr2_body,
        out_type=jax.ShapeDtypeStruct((N_NODES,), jnp.float32),
        mesh=mesh,
        compiler_params=pltpu.CompilerParams(use_tc_tiling_on_sc=False, needs_layout_passes=False),
        scratch_types=[
            pltpu.VMEM((N_NODES,), jnp.float32),
            pltpu.VMEM((ROWS_L2, CHUNK), jnp.int32),
            pltpu.VMEM((ROWS_L2, CHUNK), jnp.int32),
            pltpu.VMEM((ROWS_L2, CHUNK), jnp.float32),
            pltpu.VMEM((640, D_HID), jnp.float32),
            pltpu.VMEM((640, D_HID), jnp.float32),
            pltpu.VMEM((640, D_HID), jnp.float32),
            pltpu.VMEM((16, 16), jnp.float32),
            pltpu.VMEM((4, 16), jnp.float32),
            pltpu.VMEM((640,), jnp.float32),
            pltpu.VMEM((640,), jnp.float32),
            pltpu.VMEM((NBUF, CHUNK), jnp.float32),
            pltpu.VMEM((640,), jnp.float32),
            pltpu.VMEM_SHARED((N_NODES,), jnp.float32),
            pltpu.VMEM_SHARED((N_NODES,), jnp.float32),
            pltpu.SemaphoreType.DMA,
            [pltpu.SemaphoreType.DMA] * NBUF,
        ],
    )(part, r1, b1, wr2, wo2, b2, eiR, ewR)


def kernel(x, edge_index, edge_weight, W_rel1, b_rel1, W_root1,
           W_rel2, b_rel2, W_root2):
    npad = EPAD - N_EDGES
    ei = edge_index.astype(jnp.int32)
    eiR = jnp.concatenate(
        [ei, jnp.zeros((2, npad), jnp.int32)], axis=1).reshape(
            2, ROWS, CHUNK)
    ewR = jnp.concatenate(
        [edge_weight, jnp.zeros((npad,), jnp.float32)]).reshape(ROWS, CHUNK)

    y1, r1 = _proj1(x, W_rel1, W_root1)
    part = _sc_aggr1(y1, eiR, ewR)
    out = _sc_aggr2(part, r1, b_rel1,
                    W_rel2.reshape(D_HID), W_root2.reshape(D_HID),
                    jnp.broadcast_to(b_rel2, (D_HID,)),
                    eiR, ewR)
    return out.reshape(N_NODES, 1)


# speedup vs baseline: 30.1400x; 1.0238x over previous
import functools

import jax
import jax.numpy as jnp
from jax import lax
from jax.experimental import pallas as pl
from jax.experimental.pallas import tpu as pltpu
from jax.experimental.pallas import tpu_sc as plsc

N_NODES = 10000
N_EDGES = 320000
D_IN = 128
D_HID = 16

CHUNK = 128
ROWS = 2560
ROWS_L1 = ROWS // 32
ROWS_L2 = ROWS // 16
EPAD = ROWS * CHUNK
NBUF = 8


def _mm_kernel(x_ref, wr_ref, wo_ref, y1_ref, r1_ref):
    xb = x_ref[...]
    dn = (((1,), (1,)), ((), ()))
    y1_ref[...] = lax.dot_general(xb, wr_ref[...], dn,
                                  preferred_element_type=jnp.float32)
    r1_ref[...] = lax.dot_general(xb, wo_ref[...], dn,
                                  preferred_element_type=jnp.float32)


def _proj1(x, W_rel1, W_root1):
    blk = 2000
    return pl.pallas_call(
        _mm_kernel,
        grid=(N_NODES // blk,),
        in_specs=[
            pl.BlockSpec((blk, D_IN), lambda i: (i, 0)),
            pl.BlockSpec((D_HID, D_IN), lambda i: (0, 0)),
            pl.BlockSpec((D_HID, D_IN), lambda i: (0, 0)),
        ],
        out_specs=[
            pl.BlockSpec((blk, D_HID), lambda i: (i, 0)),
            pl.BlockSpec((blk, D_HID), lambda i: (i, 0)),
        ],
        out_shape=[
            jax.ShapeDtypeStruct((N_NODES, D_HID), jnp.float32),
            jax.ShapeDtypeStruct((N_NODES, D_HID), jnp.float32),
        ],
    )(x, W_rel1, W_root1)


def _sc_aggr1_body(y1, eiR, ewR, part,
                   src_v, dst_v, ew_v, gbuf, sbuf, zbuf, aggr,
                   gsem, ssem):
    c = lax.axis_index("c")
    s = lax.axis_index("s")
    wid = c * 16 + s
    base = wid * ROWS_L1

    pltpu.async_copy(eiR.at[0, pl.ds(base, ROWS_L1)], src_v, gsem[0])
    pltpu.async_copy(eiR.at[1, pl.ds(base, ROWS_L1)], dst_v, gsem[1])
    pltpu.async_copy(ewR.at[pl.ds(base, ROWS_L1)], ew_v, gsem[2])

    def _z(i, _):
        zbuf[i, :] = jnp.zeros((16,), jnp.float32)
        return 0
    lax.fori_loop(0, 640, _z, 0)

    @pl.when(s < 15)
    def _():
        pltpu.sync_copy(zbuf, aggr.at[pl.ds(s * 640, 640)])

    @pl.when(s == 15)
    def _():
        pltpu.sync_copy(zbuf.at[pl.ds(0, 400)], aggr.at[pl.ds(9600, 400)])

    pltpu.make_async_copy(
        eiR.at[0, pl.ds(base, ROWS_L1)], src_v, gsem[0]).wait()
    pltpu.make_async_copy(
        eiR.at[1, pl.ds(base, ROWS_L1)], dst_v, gsem[1]).wait()
    pltpu.make_async_copy(
        ewR.at[pl.ds(base, ROWS_L1)], ew_v, gsem[2]).wait()
    plsc.subcore_barrier()

    n_rounds = ROWS_L1 // NBUF
    for jj in range(NBUF):
        pltpu.async_copy(y1.at[src_v.at[jj]], gbuf.at[jj], gsem[jj])

    def _round(r, _):
        for jj in range(NBUF):
            j = r * NBUF + jj
            pltpu.make_async_copy(
                y1.at[src_v.at[j]], gbuf.at[jj], gsem[jj]).wait()

            @pl.when(r > 0)
            def _():
                pltpu.make_async_copy(
                    sbuf.at[jj], aggr.at[dst_v.at[j]], ssem[jj]).wait()

            for k in range(CHUNK // 16):
                w = ew_v[j, pl.ds(16 * k, 16)]
                for e in range(16):
                    sbuf[jj, 16 * k + e, :] = gbuf[jj, 16 * k + e, :] * w[e]

            @pl.when(r < n_rounds - 1)
            def _():
                pltpu.async_copy(
                    y1.at[src_v.at[j + NBUF]], gbuf.at[jj], gsem[jj])

            pltpu.async_copy(
                sbuf.at[jj], aggr.at[dst_v.at[j]], ssem[jj], add=True)
        return 0
    lax.fori_loop(0, n_rounds, _round, 0)
    for jj in range(NBUF):
        pltpu.make_async_copy(
            sbuf.at[jj],
            aggr.at[dst_v.at[(n_rounds - 1) * NBUF + jj]], ssem[jj]).wait()
    plsc.subcore_barrier()

    @pl.when(s < 15)
    def _():
        pltpu.sync_copy(aggr.at[pl.ds(s * 640, 640)],
                        part.at[c, pl.ds(s * 640, 640)])

    @pl.when(s == 15)
    def _():
        pltpu.sync_copy(aggr.at[pl.ds(9600, 400)],
                        part.at[c, pl.ds(9600, 400)])


def _sc_aggr1(y1, eiR, ewR):
    mesh = plsc.VectorSubcoreMesh(core_axis_name="c", subcore_axis_name="s")
    return pl.kernel(
        _sc_aggr1_body,
        out_type=jax.ShapeDtypeStruct((2, N_NODES, D_HID), jnp.float32),
        mesh=mesh,
        compiler_params=pltpu.CompilerParams(use_tc_tiling_on_sc=False, needs_layout_passes=False),
        scratch_types=[
            pltpu.VMEM((ROWS_L1, CHUNK), jnp.int32),
            pltpu.VMEM((ROWS_L1, CHUNK), jnp.int32),
            pltpu.VMEM((ROWS_L1, CHUNK), jnp.float32),
            pltpu.VMEM((NBUF, CHUNK, D_HID), jnp.float32),
            pltpu.VMEM((NBUF, CHUNK, D_HID), jnp.float32),
            pltpu.VMEM((640, D_HID), jnp.float32),
            pltpu.VMEM_SHARED((N_NODES, D_HID), jnp.float32),
            [pltpu.SemaphoreType.DMA] * NBUF,
            [pltpu.SemaphoreType.DMA] * NBUF,
        ],
    )(y1, eiR, ewR)


def _sc_aggr2_body(part, r1, b1, wr2, wo2, b2, eiR, ewR, out,
                   y2_v, src_v, dst_v, ew_v, p0_v, p1_v, r1_v, ht,
                   cst_v, y2loc, r2loc, pbuf, ibuf, y2sh, aggr,
                   esem, ssem):
    c = lax.axis_index("c")
    s = lax.axis_index("s")

    @pl.when(c == 0)
    def _():
        pltpu.async_copy(eiR.at[0, pl.ds(s * ROWS_L2, ROWS_L2)], src_v, esem)
        pltpu.async_copy(eiR.at[1, pl.ds(s * ROWS_L2, ROWS_L2)], dst_v, esem)
        pltpu.async_copy(ewR.at[pl.ds(s * ROWS_L2, ROWS_L2)], ew_v, esem)

        pltpu.async_copy(b1, cst_v.at[0], ssem[0])
        pltpu.async_copy(wr2, cst_v.at[1], ssem[1])
        pltpu.async_copy(wo2, cst_v.at[2], ssem[2])
        pltpu.async_copy(b2, cst_v.at[3], ssem[3])
        pltpu.make_async_copy(b1, cst_v.at[0], ssem[0]).wait()
        pltpu.make_async_copy(wr2, cst_v.at[1], ssem[1]).wait()
        pltpu.make_async_copy(wo2, cst_v.at[2], ssem[2]).wait()
        pltpu.make_async_copy(b2, cst_v.at[3], ssem[3]).wait()
        b1v = cst_v[0, :]
        wr2v = cst_v[1, :]
        wo2v = cst_v[2, :]
        b2s = cst_v[3, :][0]

        base = s * 640
        nn_s = [640, 400]

        def _dense(nn):
            pltpu.async_copy(part.at[0, pl.ds(base, nn)],
                             p0_v.at[pl.ds(0, nn)], ssem[4])
            pltpu.async_copy(part.at[1, pl.ds(base, nn)],
                             p1_v.at[pl.ds(0, nn)], ssem[5])
            pltpu.async_copy(r1.at[pl.ds(base, nn)],
                             r1_v.at[pl.ds(0, nn)], ssem[6])
            pltpu.make_async_copy(part.at[0, pl.ds(base, nn)],
                                  p0_v.at[pl.ds(0, nn)], ssem[4]).wait()
            pltpu.make_async_copy(part.at[1, pl.ds(base, nn)],
                                  p1_v.at[pl.ds(0, nn)], ssem[5]).wait()
            pltpu.make_async_copy(r1.at[pl.ds(base, nn)],
                                  r1_v.at[pl.ds(0, nn)], ssem[6]).wait()
            lanes = lax.broadcasted_iota(jnp.int32, (16,), 0)

            def _grp(g, _):
                for n in range(16):
                    h = jnp.maximum(
                        p0_v[16 * g + n, :] + p1_v[16 * g + n, :]
                        + r1_v[16 * g + n, :] + b1v, 0.0)
                    plsc.store_scatter(
                        ht, [lanes, jnp.full((16,), n, jnp.int32)], h)
                acc = jnp.zeros((16,), jnp.float32)
                acc2 = jnp.zeros((16,), jnp.float32)
                for f in range(16):
                    col = ht[f, :]
                    acc = acc + col * wr2v[f]
                    acc2 = acc2 + col * wo2v[f]
                y2loc[pl.ds(16 * g, 16)] = acc
                r2loc[pl.ds(16 * g, 16)] = acc2 + b2s
                return 0
            lax.fori_loop(0, nn // 16, _grp, 0)
            pltpu.sync_copy(y2loc.at[pl.ds(0, nn)],
                            y2sh.at[pl.ds(base, nn)])
            pltpu.sync_copy(r2loc.at[pl.ds(0, nn)],
                            aggr.at[pl.ds(base, nn)])

        @pl.when(s < 15)
        def _():
            _dense(nn_s[0])

        @pl.when(s == 15)
        def _():
            _dense(nn_s[1])

        plsc.subcore_barrier()
        pltpu.sync_copy(y2sh, y2_v)

        pltpu.make_async_copy(
            eiR.at[0, pl.ds(s * ROWS_L2, ROWS_L2)], src_v, esem).wait()
        pltpu.make_async_copy(
            eiR.at[1, pl.ds(s * ROWS_L2, ROWS_L2)], dst_v, esem).wait()
        pltpu.make_async_copy(
            ewR.at[pl.ds(s * ROWS_L2, ROWS_L2)], ew_v, esem).wait()

        n_rounds = ROWS_L2 // NBUF

        def _round(r, _):
            for jj in range(NBUF):
                j = r * NBUF + jj

                @pl.when(r > 0)
                def _():
                    pltpu.make_async_copy(
                        pbuf.at[jj], aggr.at[dst_v.at[j]], ssem[jj]).wait()

                for k in range(CHUNK // 16):
                    idx = src_v[j, pl.ds(16 * k, 16)]
                    vals = plsc.load_gather(y2_v, [idx])
                    w = ew_v[j, pl.ds(16 * k, 16)]
                    pbuf[jj, pl.ds(16 * k, 16)] = vals * w
                pltpu.async_copy(
                    pbuf.at[jj], aggr.at[dst_v.at[j]], ssem[jj], add=True)
            return 0
        lax.fori_loop(0, n_rounds, _round, 0)
        for jj in range(NBUF):
            pltpu.make_async_copy(
                pbuf.at[jj],
                aggr.at[dst_v.at[(n_rounds - 1) * NBUF + jj]],
                ssem[jj]).wait()
        plsc.subcore_barrier()

        @pl.when(s < 15)
        def _():
            pltpu.sync_copy(aggr.at[pl.ds(s * 640, 640)],
                            out.at[pl.ds(s * 640, 640)])

        @pl.when(s == 15)
        def _():
            pltpu.sync_copy(aggr.at[pl.ds(9600, 400)],
                            out.at[pl.ds(9600, 400)])


def _sc_aggr2(part, r1, b1, wr2, wo2, b2, eiR, ewR):
    mesh = plsc.VectorSubcoreMesh(core_axis_name="c", subcore_axis_name="s")
    return pl.kernel(
        _sc_aggr2_body,
        out_type=jax.ShapeDtypeStruct((N_NODES,), jnp.float32),
        mesh=mesh,
        compiler_params=pltpu.CompilerParams(use_tc_tiling_on_sc=False, needs_layout_passes=False),
        scratch_types=[
            pltpu.VMEM((N_NODES,), jnp.float32),
            pltpu.VMEM((ROWS_L2, CHUNK), jnp.int32),
            pltpu.VMEM((ROWS_L2, CHUNK), jnp.int32),
            pltpu.VMEM((ROWS_L2, CHUNK), jnp.float32),
            pltpu.VMEM((640, D_HID), jnp.float32),
            pltpu.VMEM((640, D_HID), jnp.float32),
            pltpu.VMEM((640, D_HID), jnp.float32),
            pltpu.VMEM((16, 16), jnp.float32),
            pltpu.VMEM((4, 16), jnp.float32),
            pltpu.VMEM((640,), jnp.float32),
            pltpu.VMEM((640,), jnp.float32),
            pltpu.VMEM((NBUF, CHUNK), jnp.float32),
            pltpu.VMEM((640,), jnp.float32),
            pltpu.VMEM_SHARED((N_NODES,), jnp.float32),
            pltpu.VMEM_SHARED((N_NODES,), jnp.float32),
            pltpu.SemaphoreType.DMA,
            [pltpu.SemaphoreType.DMA] * NBUF,
        ],
    )(part, r1, b1, wr2, wo2, b2, eiR, ewR)


def kernel(x, edge_index, edge_weight, W_rel1, b_rel1, W_root1,
           W_rel2, b_rel2, W_root2):
    npad = EPAD - N_EDGES
    ei = edge_index.astype(jnp.int32)
    eiR = jnp.concatenate(
        [ei, jnp.zeros((2, npad), jnp.int32)], axis=1).reshape(
            2, ROWS, CHUNK)
    ewR = jnp.concatenate(
        [edge_weight, jnp.zeros((npad,), jnp.float32)]).reshape(ROWS, CHUNK)

    y1, r1 = _proj1(x, W_rel1, W_root1)
    part = _sc_aggr1(y1, eiR, ewR)
    out = _sc_aggr2(part, r1, b_rel1,
                    W_rel2.reshape(D_HID), W_root2.reshape(D_HID),
                    jnp.broadcast_to(b_rel2, (D_HID,)),
                    eiR, ewR)
    return out.reshape(N_NODES, 1)


# speedup vs baseline: 30.2833x; 1.0048x over previous
import functools

import jax
import jax.numpy as jnp
from jax import lax
from jax.experimental import pallas as pl
from jax.experimental.pallas import tpu as pltpu
from jax.experimental.pallas import tpu_sc as plsc

N_NODES = 10000
N_EDGES = 320000
D_IN = 128
D_HID = 16

CHUNK = 128
ROWS = 2560
ROWS_L1 = ROWS // 32
ROWS_L2 = ROWS // 16
EPAD = ROWS * CHUNK
NBUF = 8


def _mm_kernel(x_ref, wr_ref, wo_ref, y1_ref, r1_ref):
    xb = x_ref[...]
    dn = (((1,), (1,)), ((), ()))
    y1_ref[...] = lax.dot_general(xb, wr_ref[...], dn,
                                  preferred_element_type=jnp.float32)
    r1_ref[...] = lax.dot_general(xb, wo_ref[...], dn,
                                  preferred_element_type=jnp.float32)


def _proj1(x, W_rel1, W_root1):
    blk = 10000
    return pl.pallas_call(
        _mm_kernel,
        grid=(N_NODES // blk,),
        in_specs=[
            pl.BlockSpec((blk, D_IN), lambda i: (i, 0)),
            pl.BlockSpec((D_HID, D_IN), lambda i: (0, 0)),
            pl.BlockSpec((D_HID, D_IN), lambda i: (0, 0)),
        ],
        out_specs=[
            pl.BlockSpec((blk, D_HID), lambda i: (i, 0)),
            pl.BlockSpec((blk, D_HID), lambda i: (i, 0)),
        ],
        out_shape=[
            jax.ShapeDtypeStruct((N_NODES, D_HID), jnp.float32),
            jax.ShapeDtypeStruct((N_NODES, D_HID), jnp.float32),
        ],
    )(x, W_rel1, W_root1)


def _sc_aggr1_body(y1, eiR, ewR, part,
                   src_v, dst_v, ew_v, gbuf, sbuf, zbuf, aggr,
                   gsem, ssem):
    c = lax.axis_index("c")
    s = lax.axis_index("s")
    wid = c * 16 + s
    base = wid * ROWS_L1

    pltpu.async_copy(eiR.at[0, pl.ds(base, ROWS_L1)], src_v, gsem[0])
    pltpu.async_copy(eiR.at[1, pl.ds(base, ROWS_L1)], dst_v, gsem[1])
    pltpu.async_copy(ewR.at[pl.ds(base, ROWS_L1)], ew_v, gsem[2])

    def _z(i, _):
        zbuf[i, :] = jnp.zeros((16,), jnp.float32)
        return 0
    lax.fori_loop(0, 640, _z, 0)

    @pl.when(s < 15)
    def _():
        pltpu.sync_copy(zbuf, aggr.at[pl.ds(s * 640, 640)])

    @pl.when(s == 15)
    def _():
        pltpu.sync_copy(zbuf.at[pl.ds(0, 400)], aggr.at[pl.ds(9600, 400)])

    pltpu.make_async_copy(
        eiR.at[0, pl.ds(base, ROWS_L1)], src_v, gsem[0]).wait()
    pltpu.make_async_copy(
        eiR.at[1, pl.ds(base, ROWS_L1)], dst_v, gsem[1]).wait()
    pltpu.make_async_copy(
        ewR.at[pl.ds(base, ROWS_L1)], ew_v, gsem[2]).wait()
    plsc.subcore_barrier()

    n_rounds = ROWS_L1 // NBUF
    for jj in range(NBUF):
        pltpu.async_copy(y1.at[src_v.at[jj]], gbuf.at[jj], gsem[jj])

    def _round(r, _):
        for jj in range(NBUF):
            j = r * NBUF + jj
            pltpu.make_async_copy(
                y1.at[src_v.at[j]], gbuf.at[jj], gsem[jj]).wait()

            @pl.when(r > 0)
            def _():
                pltpu.make_async_copy(
                    sbuf.at[jj], aggr.at[dst_v.at[j]], ssem[jj]).wait()

            for k in range(CHUNK // 16):
                w = ew_v[j, pl.ds(16 * k, 16)]
                for e in range(16):
                    sbuf[jj, 16 * k + e, :] = gbuf[jj, 16 * k + e, :] * w[e]

            @pl.when(r < n_rounds - 1)
            def _():
                pltpu.async_copy(
                    y1.at[src_v.at[j + NBUF]], gbuf.at[jj], gsem[jj])

            pltpu.async_copy(
                sbuf.at[jj], aggr.at[dst_v.at[j]], ssem[jj], add=True)
        return 0
    lax.fori_loop(0, n_rounds, _round, 0)
    for jj in range(NBUF):
        pltpu.make_async_copy(
            sbuf.at[jj],
            aggr.at[dst_v.at[(n_rounds - 1) * NBUF + jj]], ssem[jj]).wait()
    plsc.subcore_barrier()

    @pl.when(s < 15)
    def _():
        pltpu.sync_copy(aggr.at[pl.ds(s * 640, 640)],
                        part.at[c, pl.ds(s * 640, 640)])

    @pl.when(s == 15)
    def _():
        pltpu.sync_copy(aggr.at[pl.ds(9600, 400)],
                        part.at[c, pl.ds(9600, 400)])


def _sc_aggr1(y1, eiR, ewR):
    mesh = plsc.VectorSubcoreMesh(core_axis_name="c", subcore_axis_name="s")
    return pl.kernel(
        _sc_aggr1_body,
        out_type=jax.ShapeDtypeStruct((2, N_NODES, D_HID), jnp.float32),
        mesh=mesh,
        compiler_params=pltpu.CompilerParams(use_tc_tiling_on_sc=False, needs_layout_passes=False),
        scratch_types=[
            pltpu.VMEM((ROWS_L1, CHUNK), jnp.int32),
            pltpu.VMEM((ROWS_L1, CHUNK), jnp.int32),
            pltpu.VMEM((ROWS_L1, CHUNK), jnp.float32),
            pltpu.VMEM((NBUF, CHUNK, D_HID), jnp.float32),
            pltpu.VMEM((NBUF, CHUNK, D_HID), jnp.float32),
            pltpu.VMEM((640, D_HID), jnp.float32),
            pltpu.VMEM_SHARED((N_NODES, D_HID), jnp.float32),
            [pltpu.SemaphoreType.DMA] * NBUF,
            [pltpu.SemaphoreType.DMA] * NBUF,
        ],
    )(y1, eiR, ewR)


def _sc_aggr2_body(part, r1, b1, wr2, wo2, b2, eiR, ewR, out,
                   y2_v, src_v, dst_v, ew_v, p0_v, p1_v, r1_v, ht,
                   cst_v, y2loc, r2loc, pbuf, ibuf, y2sh, aggr,
                   esem, ssem):
    c = lax.axis_index("c")
    s = lax.axis_index("s")

    @pl.when(c == 0)
    def _():
        pltpu.async_copy(eiR.at[0, pl.ds(s * ROWS_L2, ROWS_L2)], src_v, esem)
        pltpu.async_copy(eiR.at[1, pl.ds(s * ROWS_L2, ROWS_L2)], dst_v, esem)
        pltpu.async_copy(ewR.at[pl.ds(s * ROWS_L2, ROWS_L2)], ew_v, esem)

        pltpu.async_copy(b1, cst_v.at[0], ssem[0])
        pltpu.async_copy(wr2, cst_v.at[1], ssem[1])
        pltpu.async_copy(wo2, cst_v.at[2], ssem[2])
        pltpu.async_copy(b2, cst_v.at[3], ssem[3])
        pltpu.make_async_copy(b1, cst_v.at[0], ssem[0]).wait()
        pltpu.make_async_copy(wr2, cst_v.at[1], ssem[1]).wait()
        pltpu.make_async_copy(wo2, cst_v.at[2], ssem[2]).wait()
        pltpu.make_async_copy(b2, cst_v.at[3], ssem[3]).wait()
        b1v = cst_v[0, :]
        wr2v = cst_v[1, :]
        wo2v = cst_v[2, :]
        b2s = cst_v[3, :][0]

        base = s * 640
        nn_s = [640, 400]

        def _dense(nn):
            pltpu.async_copy(part.at[0, pl.ds(base, nn)],
                             p0_v.at[pl.ds(0, nn)], ssem[4])
            pltpu.async_copy(part.at[1, pl.ds(base, nn)],
                             p1_v.at[pl.ds(0, nn)], ssem[5])
            pltpu.async_copy(r1.at[pl.ds(base, nn)],
                             r1_v.at[pl.ds(0, nn)], ssem[6])
            pltpu.make_async_copy(part.at[0, pl.ds(base, nn)],
                                  p0_v.at[pl.ds(0, nn)], ssem[4]).wait()
            pltpu.make_async_copy(part.at[1, pl.ds(base, nn)],
                                  p1_v.at[pl.ds(0, nn)], ssem[5]).wait()
            pltpu.make_async_copy(r1.at[pl.ds(base, nn)],
                                  r1_v.at[pl.ds(0, nn)], ssem[6]).wait()
            lanes = lax.broadcasted_iota(jnp.int32, (16,), 0)

            def _grp(g, _):
                for n in range(16):
                    h = jnp.maximum(
                        p0_v[16 * g + n, :] + p1_v[16 * g + n, :]
                        + r1_v[16 * g + n, :] + b1v, 0.0)
                    plsc.store_scatter(
                        ht, [lanes, jnp.full((16,), n, jnp.int32)], h)
                acc = jnp.zeros((16,), jnp.float32)
                acc2 = jnp.zeros((16,), jnp.float32)
                for f in range(16):
                    col = ht[f, :]
                    acc = acc + col * wr2v[f]
                    acc2 = acc2 + col * wo2v[f]
                y2loc[pl.ds(16 * g, 16)] = acc
                r2loc[pl.ds(16 * g, 16)] = acc2 + b2s
                return 0
            lax.fori_loop(0, nn // 16, _grp, 0)
            pltpu.sync_copy(y2loc.at[pl.ds(0, nn)],
                            y2sh.at[pl.ds(base, nn)])
            pltpu.sync_copy(r2loc.at[pl.ds(0, nn)],
                            aggr.at[pl.ds(base, nn)])

        @pl.when(s < 15)
        def _():
            _dense(nn_s[0])

        @pl.when(s == 15)
        def _():
            _dense(nn_s[1])

        plsc.subcore_barrier()
        pltpu.sync_copy(y2sh, y2_v)

        pltpu.make_async_copy(
            eiR.at[0, pl.ds(s * ROWS_L2, ROWS_L2)], src_v, esem).wait()
        pltpu.make_async_copy(
            eiR.at[1, pl.ds(s * ROWS_L2, ROWS_L2)], dst_v, esem).wait()
        pltpu.make_async_copy(
            ewR.at[pl.ds(s * ROWS_L2, ROWS_L2)], ew_v, esem).wait()

        n_rounds = ROWS_L2 // NBUF

        def _round(r, _):
            for jj in range(NBUF):
                j = r * NBUF + jj

                @pl.when(r > 0)
                def _():
                    pltpu.make_async_copy(
                        pbuf.at[jj], aggr.at[dst_v.at[j]], ssem[jj]).wait()

                for k in range(CHUNK // 16):
                    idx = src_v[j, pl.ds(16 * k, 16)]
                    vals = plsc.load_gather(y2_v, [idx])
                    w = ew_v[j, pl.ds(16 * k, 16)]
                    pbuf[jj, pl.ds(16 * k, 16)] = vals * w
                pltpu.async_copy(
                    pbuf.at[jj], aggr.at[dst_v.at[j]], ssem[jj], add=True)
            return 0
        lax.fori_loop(0, n_rounds, _round, 0)
        for jj in range(NBUF):
            pltpu.make_async_copy(
                pbuf.at[jj],
                aggr.at[dst_v.at[(n_rounds - 1) * NBUF + jj]],
                ssem[jj]).wait()
        plsc.subcore_barrier()

        @pl.when(s < 15)
        def _():
            pltpu.sync_copy(aggr.at[pl.ds(s * 640, 640)],
                            out.at[pl.ds(s * 640, 640)])

        @pl.when(s == 15)
        def _():
            pltpu.sync_copy(aggr.at[pl.ds(9600, 400)],
                            out.at[pl.ds(9600, 400)])


def _sc_aggr2(part, r1, b1, wr2, wo2, b2, eiR, ewR):
    mesh = plsc.VectorSubcoreMesh(core_axis_name="c", subcore_axis_name="s")
    return pl.kernel(
        _sc_aggr2_body,
        out_type=jax.ShapeDtypeStruct((N_NODES,), jnp.float32),
        mesh=mesh,
        compiler_params=pltpu.CompilerParams(use_tc_tiling_on_sc=False, needs_layout_passes=False),
        scratch_types=[
            pltpu.VMEM((N_NODES,), jnp.float32),
            pltpu.VMEM((ROWS_L2, CHUNK), jnp.int32),
            pltpu.VMEM((ROWS_L2, CHUNK), jnp.int32),
            pltpu.VMEM((ROWS_L2, CHUNK), jnp.float32),
            pltpu.VMEM((640, D_HID), jnp.float32),
            pltpu.VMEM((640, D_HID), jnp.float32),
            pltpu.VMEM((640, D_HID), jnp.float32),
            pltpu.VMEM((16, 16), jnp.float32),
            pltpu.VMEM((4, 16), jnp.float32),
            pltpu.VMEM((640,), jnp.float32),
            pltpu.VMEM((640,), jnp.float32),
            pltpu.VMEM((NBUF, CHUNK), jnp.float32),
            pltpu.VMEM((640,), jnp.float32),
            pltpu.VMEM_SHARED((N_NODES,), jnp.float32),
            pltpu.VMEM_SHARED((N_NODES,), jnp.float32),
            pltpu.SemaphoreType.DMA,
            [pltpu.SemaphoreType.DMA] * NBUF,
        ],
    )(part, r1, b1, wr2, wo2, b2, eiR, ewR)


def kernel(x, edge_index, edge_weight, W_rel1, b_rel1, W_root1,
           W_rel2, b_rel2, W_root2):
    npad = EPAD - N_EDGES
    ei = edge_index.astype(jnp.int32)
    eiR = jnp.concatenate(
        [ei, jnp.zeros((2, npad), jnp.int32)], axis=1).reshape(
            2, ROWS, CHUNK)
    ewR = jnp.concatenate(
        [edge_weight, jnp.zeros((npad,), jnp.float32)]).reshape(ROWS, CHUNK)

    y1, r1 = _proj1(x, W_rel1, W_root1)
    part = _sc_aggr1(y1, eiR, ewR)
    out = _sc_aggr2(part, r1, b_rel1,
                    W_rel2.reshape(D_HID), W_root2.reshape(D_HID),
                    jnp.broadcast_to(b_rel2, (D_HID,)),
                    eiR, ewR)
    return out.reshape(N_NODES, 1)
